# baseline (speedup 1.0000x reference)
import jax
import jax.numpy as jnp
from jax import lax
from jax.experimental import pallas as pl
from jax.experimental.pallas import tpu as pltpu
from jax.experimental.pallas import tpu_sc as plsc

B, G, D = 16, 2048, 128
BSC = 8
QG = G // 4
CH = 256
NCH = QG // CH
NL = 16
ND = D // NL


def _sc_body(h_hbm, adj2_hbm, phone_hbm, outbf_hbm, outaf_hbm,
             phone_v, row_v, h_buf0, h_buf1, accbf_v, accaf_v, tmp_v, shared,
             sem_r, sem0, sem1):
    c = lax.axis_index("c")
    s = lax.axis_index("s")
    b = c * 4 + s // 4
    q = s % 4
    bl = s // 4
    g0 = q * QG

    pltpu.sync_copy(phone_hbm, phone_v.at[pl.ds(0, B)])
    p = phone_v[pl.ds(b, NL)][0]

    row_cp = pltpu.make_async_copy(
        adj2_hbm.at[b * G + p, pl.ds(g0, QG)], row_v, sem_r)
    row_cp.start()

    def h_src(ci):
        return h_hbm.at[b, pl.ds(g0 + ci * CH, CH)]

    pltpu.make_async_copy(h_src(0), h_buf0, sem0).start()
    pltpu.make_async_copy(h_src(1), h_buf1, sem1).start()

    zero = jnp.zeros((NL,), jnp.float32)
    for dv in range(ND):
        accbf_v[pl.ds(dv * NL, NL)] = zero
        accaf_v[pl.ds(dv * NL, NL)] = zero

    row_cp.wait()
    lanes = lax.broadcasted_iota(jnp.int32, (NL,), 0)

    def compute(buf, ci):
        gb = g0 + ci * CH
        lb = ci * CH

        def grp_single(acc_ref):
            def body(gi, _):
                base = gi * NL
                w16 = row_v[pl.ds(lb + base, NL)]
                accs = [acc_ref[pl.ds(dv * NL, NL)] for dv in range(ND)]
                for j in range(NL):
                    w = w16[j]
                    for dv in range(ND):
                        hv = buf[base + j, pl.ds(dv * NL, NL)]
                        accs[dv] = accs[dv] + w * hv
                for dv in range(ND):
                    acc_ref[pl.ds(dv * NL, NL)] = accs[dv]
                return 0
            return body

        def grp_mixed(gi, _):
            base = gi * NL
            w16 = row_v[pl.ds(lb + base, NL)]
            gidx = gb + base + lanes
            wbf16 = jnp.where(gidx < p, w16, 0.0)
            waf16 = jnp.where(gidx > p, w16, 0.0)
            abf = [accbf_v[pl.ds(dv * NL, NL)] for dv in range(ND)]
            aaf = [accaf_v[pl.ds(dv * NL, NL)] for dv in range(ND)]
            for j in range(NL):
                wbf = wbf16[j]
                waf = waf16[j]
                for dv in range(ND):
                    hv = buf[base + j, pl.ds(dv * NL, NL)]
                    abf[dv] = abf[dv] + wbf * hv
                    aaf[dv] = aaf[dv] + waf * hv
            for dv in range(ND):
                accbf_v[pl.ds(dv * NL, NL)] = abf[dv]
                accaf_v[pl.ds(dv * NL, NL)] = aaf[dv]
            return 0

        full_bf = gb + CH <= p
        full_af = gb > p

        @pl.when(full_bf)
        def _():
            lax.fori_loop(0, CH // NL, grp_single(accbf_v), 0)

        @pl.when(full_af)
        def _():
            lax.fori_loop(0, CH // NL, grp_single(accaf_v), 0)

        @pl.when(jnp.logical_not(jnp.logical_or(full_bf, full_af)))
        def _():
            lax.fori_loop(0, CH // NL, grp_mixed, 0)

    pltpu.make_async_copy(h_src(0), h_buf0, sem0).wait()
    compute(h_buf0, 0)
    pltpu.make_async_copy(h_src(1), h_buf1, sem1).wait()
    compute(h_buf1, 1)

    @pl.when(q > 0)
    def _publish():
        pltpu.sync_copy(accbf_v, shared.at[bl, q - 1, 0])
        pltpu.sync_copy(accaf_v, shared.at[bl, q - 1, 1])

    plsc.subcore_barrier()

    @pl.when(q == 0)
    def _combine():
        for k in range(3):
            pltpu.sync_copy(shared.at[bl, k, 0], tmp_v)
            for dv in range(ND):
                ds = pl.ds(dv * NL, NL)
                accbf_v[ds] = accbf_v[ds] + tmp_v[ds]
            pltpu.sync_copy(shared.at[bl, k, 1], tmp_v)
            for dv in range(ND):
                ds = pl.ds(dv * NL, NL)
                accaf_v[ds] = accaf_v[ds] + tmp_v[ds]
        pltpu.sync_copy(accbf_v, outbf_hbm.at[b])
        pltpu.sync_copy(accaf_v, outaf_hbm.at[b])


def _tc_body(phone_ref, adj_row_ref, h_ref, obf_ref, oaf_ref):
    i = pl.program_id(0)
    p = phone_ref[BSC + i]
    row = adj_row_ref[0]
    j = lax.broadcasted_iota(jnp.int32, (1, G), 1)
    wbf = jnp.where(j < p, row, 0.0)
    waf = jnp.where(j > p, row, 0.0)
    w = jnp.concatenate([wbf, waf], axis=0)
    r = lax.dot_general(w, h_ref[0], (((1,), (0,)), ((), ())),
                        preferred_element_type=jnp.float32)
    obf_ref[0, 0] = r[0]
    oaf_ref[0, 0] = r[1]


def kernel(h, adj_c, phone_set):
    adj2 = adj_c.reshape(B * G, G)
    phone = phone_set.astype(jnp.int32)

    sc_run = pl.kernel(
        _sc_body,
        out_type=(
            jax.ShapeDtypeStruct((BSC, D), jnp.float32),
            jax.ShapeDtypeStruct((BSC, D), jnp.float32),
        ),
        mesh=plsc.VectorSubcoreMesh(core_axis_name="c", subcore_axis_name="s"),
        scratch_types=(
            pltpu.VMEM((2 * NL,), jnp.int32),
            pltpu.VMEM((QG,), jnp.float32),
            pltpu.VMEM((CH, D), jnp.float32),
            pltpu.VMEM((CH, D), jnp.float32),
            pltpu.VMEM((D,), jnp.float32),
            pltpu.VMEM((D,), jnp.float32),
            pltpu.VMEM((D,), jnp.float32),
            pltpu.VMEM_SHARED((4, 3, 2, D), jnp.float32),
            pltpu.SemaphoreType.DMA,
            pltpu.SemaphoreType.DMA,
            pltpu.SemaphoreType.DMA,
        ),
    )
    sc_bf, sc_af = sc_run(h, adj2, phone)

    grid_spec = pltpu.PrefetchScalarGridSpec(
        num_scalar_prefetch=1,
        grid=(B - BSC,),
        in_specs=[
            pl.BlockSpec((1, 1, G),
                         lambda i, ph: ((BSC + i) * G + ph[BSC + i], 0, 0)),
            pl.BlockSpec((1, G, D), lambda i, ph: (BSC + i, 0, 0)),
        ],
        out_specs=[
            pl.BlockSpec((1, 1, D), lambda i, ph: (i, 0, 0)),
            pl.BlockSpec((1, 1, D), lambda i, ph: (i, 0, 0)),
        ],
    )
    tc_bf, tc_af = pl.pallas_call(
        _tc_body,
        grid_spec=grid_spec,
        out_shape=(
            jax.ShapeDtypeStruct((B - BSC, 1, D), jnp.float32),
            jax.ShapeDtypeStruct((B - BSC, 1, D), jnp.float32),
        ),
    )(phone, adj_c.reshape(B * G, 1, G), h)
    tc_bf = tc_bf.reshape(B - BSC, D)
    tc_af = tc_af.reshape(B - BSC, D)

    g_bf = jnp.concatenate([sc_bf, tc_bf], axis=0)
    g_af = jnp.concatenate([sc_af, tc_af], axis=0)
    return (g_bf, g_af)


# speedup vs baseline: 1.0109x; 1.0109x over previous
import jax
import jax.numpy as jnp
from jax import lax
from jax.experimental import pallas as pl
from jax.experimental.pallas import tpu as pltpu

B, G, D = 16, 2048, 128


def _tc_body(phone_ref, adj_row_ref, h_ref, obf_ref, oaf_ref):
    i = pl.program_id(0)
    p = phone_ref[i]
    row = adj_row_ref[0]
    j = lax.broadcasted_iota(jnp.int32, (1, G), 1)
    wbf = jnp.where(j < p, row, 0.0)
    waf = jnp.where(j > p, row, 0.0)
    w = jnp.concatenate([wbf, waf], axis=0)
    r = lax.dot_general(w, h_ref[0], (((1,), (0,)), ((), ())),
                        preferred_element_type=jnp.float32)
    obf_ref[0, 0] = r[0]
    oaf_ref[0, 0] = r[1]


def kernel(h, adj_c, phone_set):
    phone = phone_set.astype(jnp.int32)

    grid_spec = pltpu.PrefetchScalarGridSpec(
        num_scalar_prefetch=1,
        grid=(B,),
        in_specs=[
            pl.BlockSpec((1, 1, G),
                         lambda i, ph: (i * G + ph[i], 0, 0)),
            pl.BlockSpec((1, G, D), lambda i, ph: (i, 0, 0)),
        ],
        out_specs=[
            pl.BlockSpec((1, 1, D), lambda i, ph: (i, 0, 0)),
            pl.BlockSpec((1, 1, D), lambda i, ph: (i, 0, 0)),
        ],
    )
    g_bf, g_af = pl.pallas_call(
        _tc_body,
        grid_spec=grid_spec,
        out_shape=(
            jax.ShapeDtypeStruct((B, 1, D), jnp.float32),
            jax.ShapeDtypeStruct((B, 1, D), jnp.float32),
        ),
    )(phone, adj_c.reshape(B * G, 1, G), h)
    return (g_bf.reshape(B, D), g_af.reshape(B, D))


# speedup vs baseline: 62.3667x; 61.6919x over previous
import jax
import jax.numpy as jnp
from jax import lax
from jax.experimental import pallas as pl
from jax.experimental.pallas import tpu as pltpu

B, G, D = 16, 2048, 128


def _tc_body(phone_ref, adj_grp_ref, h_ref, obf_ref, oaf_ref):
    i = pl.program_id(0)
    p = phone_ref[i]
    off = p % 8
    grp = adj_grp_ref[...]
    rsel = lax.broadcasted_iota(jnp.int32, (8, G), 0)
    row = jnp.sum(jnp.where(rsel == off, grp, 0.0), axis=0, keepdims=True)
    j = lax.broadcasted_iota(jnp.int32, (1, G), 1)
    wbf = jnp.where(j < p, row, 0.0)
    waf = jnp.where(j > p, row, 0.0)
    w = jnp.concatenate([wbf, waf], axis=0)
    r = lax.dot_general(w, h_ref[0], (((1,), (0,)), ((), ())),
                        preferred_element_type=jnp.float32)
    obf_ref[0, 0] = r[0]
    oaf_ref[0, 0] = r[1]


def kernel(h, adj_c, phone_set):
    phone = phone_set.astype(jnp.int32)

    grid_spec = pltpu.PrefetchScalarGridSpec(
        num_scalar_prefetch=1,
        grid=(B,),
        in_specs=[
            pl.BlockSpec((8, G), lambda i, ph: (i * (G // 8) + ph[i] // 8, 0)),
            pl.BlockSpec((1, G, D), lambda i, ph: (i, 0, 0)),
        ],
        out_specs=[
            pl.BlockSpec((1, 1, D), lambda i, ph: (i, 0, 0)),
            pl.BlockSpec((1, 1, D), lambda i, ph: (i, 0, 0)),
        ],
    )
    g_bf, g_af = pl.pallas_call(
        _tc_body,
        grid_spec=grid_spec,
        out_shape=(
            jax.ShapeDtypeStruct((B, 1, D), jnp.float32),
            jax.ShapeDtypeStruct((B, 1, D), jnp.float32),
        ),
    )(phone, adj_c.reshape(B * G, G), h)
    return (g_bf.reshape(B, D), g_af.reshape(B, D))


# speedup vs baseline: 62.6733x; 1.0049x over previous
import jax
import jax.numpy as jnp
from jax import lax
from jax.experimental import pallas as pl
from jax.experimental.pallas import tpu as pltpu

B, G, D = 16, 2048, 128


def _tc_body(phone_ref, adj_grp_ref, h_ref, obf_ref, oaf_ref):
    i = pl.program_id(0)
    p = phone_ref[i]
    off = p % 8
    grp = adj_grp_ref[...]
    rsel = lax.broadcasted_iota(jnp.int32, (8, G), 0)
    row = jnp.sum(jnp.where(rsel == off, grp, 0.0), axis=0, keepdims=True)
    j = lax.broadcasted_iota(jnp.int32, (1, G), 1)
    wbf = jnp.where(j < p, row, 0.0)
    waf = jnp.where(j > p, row, 0.0)
    w = jnp.concatenate([wbf, waf], axis=0)
    r = lax.dot_general(w, h_ref[0], (((1,), (0,)), ((), ())),
                        preferred_element_type=jnp.float32)
    obf_ref[0, 0] = r[0]
    oaf_ref[0, 0] = r[1]


def kernel(h, adj_c, phone_set):
    phone = phone_set.astype(jnp.int32)

    grid_spec = pltpu.PrefetchScalarGridSpec(
        num_scalar_prefetch=1,
        grid=(B,),
        in_specs=[
            pl.BlockSpec((8, G), lambda i, ph: (i * (G // 8) + ph[i] // 8, 0)),
            pl.BlockSpec((1, G, D), lambda i, ph: (i, 0, 0)),
        ],
        out_specs=[
            pl.BlockSpec((1, 1, D), lambda i, ph: (i, 0, 0)),
            pl.BlockSpec((1, 1, D), lambda i, ph: (i, 0, 0)),
        ],
    )
    g_bf, g_af = pl.pallas_call(
        _tc_body,
        grid_spec=grid_spec,
        compiler_params=pltpu.CompilerParams(
            dimension_semantics=("parallel",)),
        out_shape=(
            jax.ShapeDtypeStruct((B, 1, D), jnp.float32),
            jax.ShapeDtypeStruct((B, 1, D), jnp.float32),
        ),
    )(phone, adj_c.reshape(B * G, G), h)
    return (g_bf.reshape(B, D), g_af.reshape(B, D))
